# baseline (speedup 1.0000x reference)
import jax
import jax.numpy as jnp
from jax import lax
from jax.experimental import pallas as pl
from jax.experimental.pallas import tpu as pltpu
from jax.experimental.pallas import tpu_sc as plsc

_BATCH = 16384
_D = 32
_NC = 2
_NS = 16
_NW = _NC * _NS
_BPW = _BATCH // _NW
_CHUNK = 128
_NCHUNK = _BPW // _CHUNK
_L = 16
_WAVE = 4


def _mf_body(uid_hbm, iid_hbm, pt_hbm, qt_hbm, ub_hbm, ib_hbm, out_hbm,
             uidx, iidx, pbuf, qbuf, bu_v, bi_v, out_v, sem0, sem1, semb):
    wid = lax.axis_index("s") * _NC + lax.axis_index("c")
    sems = (sem0, sem1)
    pltpu.sync_copy(uid_hbm.at[wid], uidx)
    pltpu.sync_copy(iid_hbm.at[wid], iidx)

    bias_cps = []
    for c in range(_NCHUNK):
        sl = pl.ds(c * _CHUNK, _CHUNK)
        bias_cps.append(pltpu.async_copy(ub_hbm.at[uidx.at[c]], bu_v.at[sl], semb))
        bias_cps.append(pltpu.async_copy(ib_hbm.at[iidx.at[c]], bi_v.at[sl], semb))

    def fire_wave(w):
        cps = []
        for k in range(w * _WAVE, (w + 1) * _WAVE):
            for c in range(_NCHUNK):
                sl = pl.ds(c * _CHUNK, _CHUNK)
                cps.append(pltpu.async_copy(
                    pt_hbm.at[k].at[uidx.at[c]], pbuf.at[k, sl], sems[w % 2]))
                cps.append(pltpu.async_copy(
                    qt_hbm.at[k].at[iidx.at[c]], qbuf.at[k, sl], sems[w % 2]))
        return cps

    nwaves = _D // _WAVE
    pending = fire_wave(0)
    for w in range(1, nwaves + 1):
        nxt = fire_wave(w) if w < nwaves else []
        for cp in pending:
            cp.wait()
        pending = nxt
    for cp in bias_cps:
        cp.wait()

    def group(g, carry):
        gb = g * _L
        acc = bu_v[pl.ds(gb, _L)] + bi_v[pl.ds(gb, _L)]
        for k in range(_D):
            acc = acc + pbuf[k, pl.ds(gb, _L)] * qbuf[k, pl.ds(gb, _L)]
        out_v[pl.ds(gb, _L)] = acc
        return carry

    lax.fori_loop(0, _BPW // _L, group, 0)

    pltpu.sync_copy(out_v, out_hbm.at[pl.ds(wid * _BPW, _BPW)])


@jax.jit
def _mf(uid3, iid3, PT, QT, ub, ib):
    mesh = plsc.VectorSubcoreMesh(core_axis_name="c", subcore_axis_name="s")
    return pl.kernel(
        _mf_body,
        mesh=mesh,
        compiler_params=pltpu.CompilerParams(
            needs_layout_passes=False, use_tc_tiling_on_sc=False),
        out_type=jax.ShapeDtypeStruct((_BATCH,), jnp.float32),
        scratch_types=[
            pltpu.VMEM((_NCHUNK, _CHUNK), jnp.int32),
            pltpu.VMEM((_NCHUNK, _CHUNK), jnp.int32),
            pltpu.VMEM((_D, _BPW), jnp.float32),
            pltpu.VMEM((_D, _BPW), jnp.float32),
            pltpu.VMEM((_BPW,), jnp.float32),
            pltpu.VMEM((_BPW,), jnp.float32),
            pltpu.VMEM((_BPW,), jnp.float32),
            pltpu.SemaphoreType.DMA,
            pltpu.SemaphoreType.DMA,
            pltpu.SemaphoreType.DMA,
        ],
    )(uid3, iid3, PT, QT, ub, ib)


def kernel(user_id, item_id, P, Q, user_bias, item_bias):
    shape3 = (_NW, _NCHUNK, _CHUNK)
    uid3 = user_id.reshape(shape3)
    iid3 = item_id.reshape(shape3)
    ub = user_bias.reshape(-1)
    ib = item_bias.reshape(-1)
    return _mf(uid3, iid3, P.T, Q.T, ub, ib)


# speedup vs baseline: 5.7209x; 5.7209x over previous
import jax
import jax.numpy as jnp
from jax import lax
from jax.experimental import pallas as pl
from jax.experimental.pallas import tpu as pltpu
from jax.experimental.pallas import tpu_sc as plsc

_BATCH = 16384
_D = 32
_NC = 2
_NS = 16
_NW = _NC * _NS
_BPW = _BATCH // _NW
_CHUNK = 128
_NCHUNK = _BPW // _CHUNK
_L = 16


def _mf_body(uid_hbm, iid_hbm, p_hbm, q_hbm, ub_hbm, ib_hbm, out_hbm,
             uidx, iidx, prow, qrow, bu_v, bi_v, out_v, sem):
    wid = lax.axis_index("s") * _NC + lax.axis_index("c")
    pltpu.sync_copy(uid_hbm.at[wid], uidx)
    pltpu.sync_copy(iid_hbm.at[wid], iidx)

    cps = []
    for c in range(_NCHUNK):
        sl = pl.ds(c * _CHUNK, _CHUNK)
        cps.append(pltpu.async_copy(p_hbm.at[uidx.at[c]], prow.at[sl], sem))
        cps.append(pltpu.async_copy(q_hbm.at[iidx.at[c]], qrow.at[sl], sem))
        cps.append(pltpu.async_copy(ub_hbm.at[uidx.at[c]], bu_v.at[sl], sem))
        cps.append(pltpu.async_copy(ib_hbm.at[iidx.at[c]], bi_v.at[sl], sem))
    for cp in cps:
        cp.wait()

    lane = lax.broadcasted_iota(jnp.int32, (_L,), 0)

    def group(g, carry):
        base = g * _L
        acc = bu_v[pl.ds(base, _L)] + bi_v[pl.ds(base, _L)]
        row_idx = lane + base
        for k in range(_D):
            col = jnp.full((_L,), k, jnp.int32)
            pv = plsc.load_gather(prow, [row_idx, col])
            qv = plsc.load_gather(qrow, [row_idx, col])
            acc = acc + pv * qv
        out_v[pl.ds(base, _L)] = acc
        return carry

    lax.fori_loop(0, _BPW // _L, group, 0)

    pltpu.sync_copy(out_v, out_hbm.at[pl.ds(wid * _BPW, _BPW)])


@jax.jit
def _mf(uid3, iid3, P, Q, ub, ib):
    mesh = plsc.VectorSubcoreMesh(core_axis_name="c", subcore_axis_name="s")
    return pl.kernel(
        _mf_body,
        mesh=mesh,
        compiler_params=pltpu.CompilerParams(
            needs_layout_passes=False, use_tc_tiling_on_sc=False),
        out_type=jax.ShapeDtypeStruct((_BATCH,), jnp.float32),
        scratch_types=[
            pltpu.VMEM((_NCHUNK, _CHUNK), jnp.int32),
            pltpu.VMEM((_NCHUNK, _CHUNK), jnp.int32),
            pltpu.VMEM((_BPW, _D), jnp.float32),
            pltpu.VMEM((_BPW, _D), jnp.float32),
            pltpu.VMEM((_BPW,), jnp.float32),
            pltpu.VMEM((_BPW,), jnp.float32),
            pltpu.VMEM((_BPW,), jnp.float32),
            pltpu.SemaphoreType.DMA,
        ],
    )(uid3, iid3, P, Q, ub, ib)


def kernel(user_id, item_id, P, Q, user_bias, item_bias):
    uid3 = user_id.reshape(_NW, _NCHUNK, _CHUNK)
    iid3 = item_id.reshape(_NW, _NCHUNK, _CHUNK)
    ub = user_bias.reshape(-1)
    ib = item_bias.reshape(-1)
    return _mf(uid3, iid3, P, Q, ub, ib)


# speedup vs baseline: 16.3879x; 2.8646x over previous
import jax
import jax.numpy as jnp
from jax import lax
from jax.experimental import pallas as pl
from jax.experimental.pallas import tpu as pltpu
from jax.experimental.pallas import tpu_sc as plsc

_BATCH = 16384
_D = 32
_NC = 2
_NS = 16
_NW = _NC * _NS
_BPW = _BATCH // _NW
_CHUNK = 128
_NCHUNK = _BPW // _CHUNK
_L = 16
_NBUF = 8


def _mf_body(uid_hbm, iid_hbm, pt_hbm, qt_hbm, ub_hbm, ib_hbm, out_hbm,
             uidx, iidx, pblk, qblk, bu_v, bi_v, out_v,
             semp0, semp1, semp2, semp3, semp4, semp5, semp6, semp7,
             semq0, semq1, semq2, semq3, semq4, semq5, semq6, semq7,
             semb):
    wid = lax.axis_index("s") * _NC + lax.axis_index("c")
    semp = (semp0, semp1, semp2, semp3, semp4, semp5, semp6, semp7)
    semq = (semq0, semq1, semq2, semq3, semq4, semq5, semq6, semq7)

    pltpu.sync_copy(uid_hbm.at[wid], uidx)
    pltpu.sync_copy(iid_hbm.at[wid], iidx)

    bias_cps = []
    for c in range(_NCHUNK):
        sl = pl.ds(c * _CHUNK, _CHUNK)
        bias_cps.append(pltpu.async_copy(ub_hbm.at[uidx.at[sl]], bu_v.at[sl], semb))
        bias_cps.append(pltpu.async_copy(ib_hbm.at[iidx.at[sl]], bi_v.at[sl], semb))

    def fire(u, i, slot):
        ubase = pl.multiple_of((u // _CHUNK) * _CHUNK, _CHUNK)
        ibase = pl.multiple_of((i // _CHUNK) * _CHUNK, _CHUNK)
        pltpu.async_copy(pt_hbm.at[:, pl.ds(ubase, _CHUNK)], pblk.at[slot], semp[slot])
        pltpu.async_copy(qt_hbm.at[:, pl.ds(ibase, _CHUNK)], qblk.at[slot], semq[slot])

    def drain(slot):
        pltpu.make_async_copy(
            pt_hbm.at[:, pl.ds(0, _CHUNK)], pblk.at[slot], semp[slot]).wait()
        pltpu.make_async_copy(
            qt_hbm.at[:, pl.ds(0, _CHUNK)], qblk.at[slot], semq[slot]).wait()

    uvec0 = uidx[pl.ds(0, _L)]
    ivec0 = iidx[pl.ds(0, _L)]
    for e in range(_NBUF):
        fire(uvec0[e], ivec0[e], e)
    for cp in bias_cps:
        cp.wait()

    kr = lax.broadcasted_iota(jnp.int32, (_L,), 0)

    def group(g, carry):
        base = g * _L
        nbase = jnp.minimum(base + _L, _BPW - _L)
        uvec = uidx[pl.ds(base, _L)]
        ivec = iidx[pl.ds(base, _L)]
        unext = uidx[pl.ds(nbase, _L)]
        inext = iidx[pl.ds(nbase, _L)]
        acc = jnp.zeros((_L,), jnp.float32)
        for e in range(_L):
            slot = e % _NBUF
            drain(slot)
            u = uvec[e]
            i = ivec[e]
            ul = jnp.full((_L,), u % _CHUNK, jnp.int32)
            il = jnp.full((_L,), i % _CHUNK, jnp.int32)
            p0 = plsc.load_gather(pblk.at[slot], [kr, ul])
            p1 = plsc.load_gather(pblk.at[slot], [kr + _L, ul])
            q0 = plsc.load_gather(qblk.at[slot], [kr, il])
            q1 = plsc.load_gather(qblk.at[slot], [kr + _L, il])
            prod = p0 * q0 + p1 * q1
            s = jnp.sum(prod)
            acc = jnp.where(kr == e, s, acc)
            if e < _L - _NBUF:
                fire(uvec[e + _NBUF], ivec[e + _NBUF], slot)
            else:
                fire(unext[e - (_L - _NBUF)], inext[e - (_L - _NBUF)], slot)
        out_v[pl.ds(base, _L)] = (
            acc + bu_v[pl.ds(base, _L)] + bi_v[pl.ds(base, _L)])
        return carry

    lax.fori_loop(0, _BPW // _L, group, 0)

    for e in range(_NBUF):
        drain(e)

    pltpu.sync_copy(out_v, out_hbm.at[pl.ds(wid * _BPW, _BPW)])


@jax.jit
def _mf(uid2, iid2, PT, QT, ub, ib):
    mesh = plsc.VectorSubcoreMesh(core_axis_name="c", subcore_axis_name="s")
    return pl.kernel(
        _mf_body,
        mesh=mesh,
        compiler_params=pltpu.CompilerParams(needs_layout_passes=False),
        out_type=jax.ShapeDtypeStruct((_BATCH,), jnp.float32),
        scratch_types=[
            pltpu.VMEM((_BPW,), jnp.int32),
            pltpu.VMEM((_BPW,), jnp.int32),
            pltpu.VMEM((_NBUF, _D, _CHUNK), jnp.float32),
            pltpu.VMEM((_NBUF, _D, _CHUNK), jnp.float32),
            pltpu.VMEM((_BPW,), jnp.float32),
            pltpu.VMEM((_BPW,), jnp.float32),
            pltpu.VMEM((_BPW,), jnp.float32),
        ] + [pltpu.SemaphoreType.DMA] * (2 * _NBUF + 1),
    )(uid2, iid2, PT, QT, ub, ib)


def kernel(user_id, item_id, P, Q, user_bias, item_bias):
    uid2 = user_id.reshape(_NW, _BPW)
    iid2 = item_id.reshape(_NW, _BPW)
    ub = user_bias.reshape(-1)
    ib = item_bias.reshape(-1)
    return _mf(uid2, iid2, P.T, Q.T, ub, ib)


# speedup vs baseline: 22.5218x; 1.3743x over previous
import jax
import jax.numpy as jnp
from jax import lax
from jax.experimental import pallas as pl
from jax.experimental.pallas import tpu as pltpu
from jax.experimental.pallas import tpu_sc as plsc

_BATCH = 16384
_D = 32
_NC = 2
_NS = 16
_NW = _NC * _NS
_BPW = _BATCH // _NW
_CHUNK = 128
_NCHUNK = _BPW // _CHUNK
_L = 16
_NBUF = 8


def _mf_body(uid_hbm, iid_hbm, pt_hbm, qt_hbm, ub_hbm, ib_hbm, out_hbm,
             uidx, iidx, pblk, qblk, bu_v, bi_v, out_v,
             semp0, semp1, semp2, semp3, semp4, semp5, semp6, semp7,
             semq0, semq1, semq2, semq3, semq4, semq5, semq6, semq7,
             semb):
    wid = lax.axis_index("s") * _NC + lax.axis_index("c")
    semp = (semp0, semp1, semp2, semp3, semp4, semp5, semp6, semp7)
    semq = (semq0, semq1, semq2, semq3, semq4, semq5, semq6, semq7)

    pltpu.sync_copy(uid_hbm.at[wid], uidx)
    pltpu.sync_copy(iid_hbm.at[wid], iidx)

    bias_cps = []
    for c in range(_NCHUNK):
        sl = pl.ds(c * _CHUNK, _CHUNK)
        bias_cps.append(pltpu.async_copy(
            ub_hbm.at[0].at[uidx.at[sl]], bu_v.at[sl], semb))
        bias_cps.append(pltpu.async_copy(
            ib_hbm.at[0].at[iidx.at[sl]], bi_v.at[sl], semb))

    def fire(u, i, slot):
        ubase = pl.multiple_of((u // _CHUNK) * _CHUNK, _CHUNK)
        ibase = pl.multiple_of((i // _CHUNK) * _CHUNK, _CHUNK)
        pltpu.async_copy(pt_hbm.at[:, pl.ds(ubase, _CHUNK)], pblk.at[slot], semp[slot])
        pltpu.async_copy(qt_hbm.at[:, pl.ds(ibase, _CHUNK)], qblk.at[slot], semq[slot])

    def drain(slot):
        pltpu.make_async_copy(
            pt_hbm.at[:, pl.ds(0, _CHUNK)], pblk.at[slot], semp[slot]).wait()
        pltpu.make_async_copy(
            qt_hbm.at[:, pl.ds(0, _CHUNK)], qblk.at[slot], semq[slot]).wait()

    uvec0 = uidx[pl.ds(0, _L)]
    ivec0 = iidx[pl.ds(0, _L)]
    for e in range(_NBUF):
        fire(uvec0[e], ivec0[e], e)
    for cp in bias_cps:
        cp.wait()

    kr = lax.broadcasted_iota(jnp.int32, (_L,), 0)

    def group(g, carry):
        base = g * _L
        nbase = jnp.minimum(base + _L, _BPW - _L)
        uvec = uidx[pl.ds(base, _L)]
        ivec = iidx[pl.ds(base, _L)]
        unext = uidx[pl.ds(nbase, _L)]
        inext = iidx[pl.ds(nbase, _L)]
        acc = jnp.zeros((_L,), jnp.float32)
        for e in range(_L):
            slot = e % _NBUF
            drain(slot)
            u = uvec[e]
            i = ivec[e]
            ul = jnp.full((_L,), u % _CHUNK, jnp.int32)
            il = jnp.full((_L,), i % _CHUNK, jnp.int32)
            p0 = plsc.load_gather(pblk.at[slot], [kr, ul])
            p1 = plsc.load_gather(pblk.at[slot], [kr + _L, ul])
            q0 = plsc.load_gather(qblk.at[slot], [kr, il])
            q1 = plsc.load_gather(qblk.at[slot], [kr + _L, il])
            prod = p0 * q0 + p1 * q1
            s = jnp.sum(prod)
            acc = jnp.where(kr == e, s, acc)
            if e < _L - _NBUF:
                fire(uvec[e + _NBUF], ivec[e + _NBUF], slot)
            else:
                fire(unext[e - (_L - _NBUF)], inext[e - (_L - _NBUF)], slot)
        out_v[pl.ds(base, _L)] = (
            acc + bu_v[pl.ds(base, _L)] + bi_v[pl.ds(base, _L)])
        return carry

    lax.fori_loop(0, _BPW // _L, group, 0)

    for e in range(_NBUF):
        drain(e)

    pltpu.sync_copy(out_v, out_hbm.at[pl.ds(wid * _BPW, _BPW)])


@jax.jit
def _mf(uid2, iid2, PT, QT, ub, ib):
    mesh = plsc.VectorSubcoreMesh(core_axis_name="c", subcore_axis_name="s")
    return pl.kernel(
        _mf_body,
        mesh=mesh,
        compiler_params=pltpu.CompilerParams(needs_layout_passes=False),
        out_type=jax.ShapeDtypeStruct((_BATCH,), jnp.float32),
        scratch_types=[
            pltpu.VMEM((_BPW,), jnp.int32),
            pltpu.VMEM((_BPW,), jnp.int32),
            pltpu.VMEM((_NBUF, _D, _CHUNK), jnp.float32),
            pltpu.VMEM((_NBUF, _D, _CHUNK), jnp.float32),
            pltpu.VMEM((_BPW,), jnp.float32),
            pltpu.VMEM((_BPW,), jnp.float32),
            pltpu.VMEM((_BPW,), jnp.float32),
        ] + [pltpu.SemaphoreType.DMA] * (2 * _NBUF + 1),
    )(uid2, iid2, PT, QT, ub, ib)


def kernel(user_id, item_id, P, Q, user_bias, item_bias):
    uid2 = user_id.reshape(_NW, _BPW)
    iid2 = item_id.reshape(_NW, _BPW)
    return _mf(uid2, iid2, P.T, Q.T, user_bias.T, item_bias.T)
